# Initial kernel scaffold; baseline (speedup 1.0000x reference)
import jax
import jax.numpy as jnp
from jax.experimental import pallas as pl


def kernel(x, edge_index, batch, Wq1, bq1, Wk1, bk1, Wv1, bv1, Ws1, bs1, Wb1, Wq2, bq2, Wk2, bk2, Wv2, bv2, Ws2, bs2, Wb2, Wl, bl):
    raise NotImplementedError("write your pallas kernel here")


# speedup vs baseline: 12.8693x; 12.8693x over previous
import functools
import jax
import jax.numpy as jnp
from jax import lax
from jax.experimental import pallas as pl
from jax.experimental.pallas import tpu as pltpu
from jax.experimental.pallas import tpu_sc as plsc

N = 10000
E = 320000
DIN = 128
H = 3
C = 64
HC = H * C
G = 64
AW = C + 16

NCORE = 2
NSUB = 16
NTILE = NCORE * NSUB
EPT = E // NTILE
BLK = 80
NBLK = EPT // BLK
RSTRIDE = 624
RSPAN = 640
ZR = 128


def _split_heads(p):
    qs = [p[:, h * C:(h + 1) * C] for h in range(H)]
    kvs = [jnp.concatenate([p[:, HC + h * C:HC + (h + 1) * C],
                            p[:, 2 * HC + h * C:2 * HC + (h + 1) * C]], axis=1)
           for h in range(H)]
    return qs, kvs, p[:, 3 * HC:]


def _proj_body(x_ref, w_ref, b_ref, *o_refs):
    p = jnp.dot(x_ref[...], w_ref[...], preferred_element_type=jnp.float32,
            precision=lax.Precision.HIGHEST)
    p = p + b_ref[...]
    qs, kvs, xr = _split_heads(p)
    for h in range(H):
        o_refs[h][...] = qs[h]
        o_refs[H + h][...] = kvs[h]
    o_refs[2 * H][...] = xr


def _head_out_specs(rows):
    return (
        [pl.BlockSpec((rows, C), lambda i: (i, 0)) for _ in range(H)]
        + [pl.BlockSpec((rows, 2 * C), lambda i: (i, 0)) for _ in range(H)]
        + [pl.BlockSpec((rows, HC), lambda i: (i, 0))]
    )


def _head_out_shapes():
    return (
        [jax.ShapeDtypeStruct((N, C), jnp.float32) for _ in range(H)]
        + [jax.ShapeDtypeStruct((N, 2 * C), jnp.float32) for _ in range(H)]
        + [jax.ShapeDtypeStruct((N, HC), jnp.float32)]
    )


def _project(x, wcat, bcat, rows):
    din = x.shape[1]
    return pl.pallas_call(
        _proj_body,
        grid=(N // rows,),
        in_specs=[
            pl.BlockSpec((rows, din), lambda i: (i, 0)),
            pl.BlockSpec((din, 4 * HC), lambda i: (0, 0)),
            pl.BlockSpec((1, 4 * HC), lambda i: (0, 0)),
        ],
        out_specs=_head_out_specs(rows),
        out_shape=_head_out_shapes(),
    )(x, wcat, bcat)


def _finish_layer(acc6, xr, wba, wbb):
    outs = []
    for h in range(H):
        a = acc6[2 * h] + acc6[2 * h + 1]
        den = a[:, C:C + 1]
        outs.append(a[:, :C] / (den + 1e-16))
    out = jnp.concatenate(outs, axis=1)
    blog = (jnp.dot(out, wba, preferred_element_type=jnp.float32,
            precision=lax.Precision.HIGHEST)
            + jnp.dot(xr, wbb, preferred_element_type=jnp.float32,
            precision=lax.Precision.HIGHEST))
    beta = 1.0 / (1.0 + jnp.exp(-blog))
    z = beta * xr + (1.0 - beta) * out
    return jnp.where(z > 0, z, jnp.exp(z) - 1.0)


def _mid_body(a_ref, xr_ref, wba_ref, wbb_ref, w_ref, b_ref,
              *o_refs):
    h = _finish_layer(a_ref[...], xr_ref[...], wba_ref[...], wbb_ref[...])
    p = jnp.dot(h, w_ref[...], preferred_element_type=jnp.float32,
            precision=lax.Precision.HIGHEST) + b_ref[...]
    qs, kvs, xr = _split_heads(p)
    for i in range(H):
        o_refs[i][...] = qs[i]
        o_refs[H + i][...] = kvs[i]
    o_refs[2 * H][...] = xr


def _mid(acc6, xr, wba, wbb, wcat, bcat, rows):
    return pl.pallas_call(
        _mid_body,
        grid=(N // rows,),
        in_specs=[
            pl.BlockSpec((2 * H, rows, AW), lambda i: (0, i, 0)),
            pl.BlockSpec((rows, HC), lambda i: (i, 0)),
            pl.BlockSpec((HC, 1), lambda i: (0, 0)),
            pl.BlockSpec((HC, 1), lambda i: (0, 0)),
            pl.BlockSpec((HC, 4 * HC), lambda i: (0, 0)),
            pl.BlockSpec((1, 4 * HC), lambda i: (0, 0)),
        ],
        out_specs=_head_out_specs(rows),
        out_shape=_head_out_shapes(),
    )(acc6, xr, wba, wbb, wcat, bcat)


def _final_body(a_ref, xr_ref, wba_ref, wbb_ref, b_ref,
                wl_ref, bl_ref, out_ref, sum_ref, cnt_ref):
    i = pl.program_id(0)
    rows = xr_ref.shape[0]

    @pl.when(i == 0)
    def _():
        sum_ref[...] = jnp.zeros_like(sum_ref)
        cnt_ref[...] = jnp.zeros_like(cnt_ref)

    h = _finish_layer(a_ref[...], xr_ref[...], wba_ref[...], wbb_ref[...])
    bb = b_ref[...]
    mt = (jnp.broadcast_to(bb, (rows, G))
          == lax.broadcasted_iota(jnp.int32, (rows, G), 1)).astype(jnp.float32)
    dn = (((0,), (0,)), ((), ()))
    sum_ref[...] += lax.dot_general(mt, h, dn, preferred_element_type=jnp.float32,
            precision=lax.Precision.HIGHEST)
    cnt_ref[...] += lax.dot_general(mt, jnp.ones((rows, 1), jnp.float32), dn,
                                    preferred_element_type=jnp.float32,
            precision=lax.Precision.HIGHEST)

    @pl.when(i == pl.num_programs(0) - 1)
    def _():
        t = jnp.dot(sum_ref[...], wl_ref[...], preferred_element_type=jnp.float32,
            precision=lax.Precision.HIGHEST)
        out_ref[...] = t / jnp.maximum(cnt_ref[...], 1.0) + bl_ref[...]


def _final(acc6, xr, wba, wbb, batch2d, wl, blb, rows):
    return pl.pallas_call(
        _final_body,
        grid=(N // rows,),
        in_specs=[
            pl.BlockSpec((2 * H, rows, AW), lambda i: (0, i, 0)),
            pl.BlockSpec((rows, HC), lambda i: (i, 0)),
            pl.BlockSpec((HC, 1), lambda i: (0, 0)),
            pl.BlockSpec((HC, 1), lambda i: (0, 0)),
            pl.BlockSpec((rows, 1), lambda i: (i, 0)),
            pl.BlockSpec((HC, 1), lambda i: (0, 0)),
            pl.BlockSpec((1, 1), lambda i: (0, 0)),
        ],
        out_specs=pl.BlockSpec((G, 1), lambda i: (0, 0)),
        out_shape=jax.ShapeDtypeStruct((G, 1), jnp.float32),
        scratch_shapes=[
            pltpu.VMEM((G, HC), jnp.float32),
            pltpu.VMEM((G, 1), jnp.float32),
        ],
    )(acc6, xr, wba, wbb, batch2d, wl, blb)


def _edge_body(src_hbm, dst_hbm, q0_hbm, q1_hbm, q2_hbm,
               kv0_hbm, kv1_hbm, kv2_hbm, out_hbm,
               acc_sh, sidx, didx, qrows, kvrows, msg, zbuf, sem1, sem2):
    cid = lax.axis_index("c")
    sid = lax.axis_index("s")
    wid = sid * NCORE + cid

    lane = lax.broadcasted_iota(jnp.int32, (16,), 0)
    zero16 = jnp.zeros((16,), jnp.float32)

    def _zrow(r, _):
        for j in range(AW // 16):
            zbuf[r, pl.ds(j * 16, 16)] = zero16
        return 0
    lax.fori_loop(0, ZR, _zrow, 0)

    for h, (q_hbm, kv_hbm) in enumerate(
            ((q0_hbm, kv0_hbm), (q1_hbm, kv1_hbm), (q2_hbm, kv2_hbm))):
        for t in range(RSPAN // ZR):
            pltpu.sync_copy(zbuf, acc_sh.at[pl.ds(sid * RSTRIDE + t * ZR, ZR)])
        plsc.subcore_barrier()

        def _block(blk, _):
            base = wid * EPT + blk * BLK
            pltpu.sync_copy(src_hbm.at[pl.ds(base, BLK)], sidx)
            pltpu.sync_copy(dst_hbm.at[pl.ds(base, BLK)], didx)
            d1 = pltpu.async_copy(q_hbm.at[didx], qrows, sem1)
            d2 = pltpu.async_copy(kv_hbm.at[sidx], kvrows, sem2)
            d1.wait()
            d2.wait()

            def _edge(e, _):
                acc = qrows[e, pl.ds(0, 16)] * kvrows[e, pl.ds(0, 16)]
                for j in range(1, C // 16):
                    acc = acc + (qrows[e, pl.ds(j * 16, 16)]
                                 * kvrows[e, pl.ds(j * 16, 16)])
                for sh in (8, 4, 2, 1):
                    acc = acc + jnp.take(acc, lane ^ sh)
                ex = jnp.exp(acc)
                for j in range(C // 16):
                    msg[e, pl.ds(j * 16, 16)] = (
                        kvrows[e, pl.ds(C + j * 16, 16)] * ex)
                msg[e, pl.ds(C, 16)] = jnp.where(lane == 0, ex, zero16)
                return 0
            lax.fori_loop(0, BLK, _edge, 0)

            pltpu.sync_copy(msg, acc_sh.at[didx], add=True)
            return 0
        lax.fori_loop(0, NBLK, _block, 0)

        plsc.subcore_barrier()
        pltpu.sync_copy(acc_sh.at[pl.ds(sid * RSTRIDE, RSPAN)],
                        out_hbm.at[h, cid, pl.ds(sid * RSTRIDE, RSPAN)])
        plsc.subcore_barrier()


@functools.lru_cache(maxsize=None)
def _make_edge_attention():
    return pl.kernel(
        _edge_body,
        out_type=jax.ShapeDtypeStruct((H, NCORE, N, AW), jnp.float32),
        mesh=plsc.VectorSubcoreMesh(core_axis_name="c", subcore_axis_name="s"),
        scratch_types=[
            pltpu.VMEM_SHARED((N, AW), jnp.float32),
            pltpu.VMEM((BLK,), jnp.int32),
            pltpu.VMEM((BLK,), jnp.int32),
            pltpu.VMEM((BLK, C), jnp.float32),
            pltpu.VMEM((BLK, 2 * C), jnp.float32),
            pltpu.VMEM((BLK, AW), jnp.float32),
            pltpu.VMEM((ZR, AW), jnp.float32),
            pltpu.SemaphoreType.DMA,
            pltpu.SemaphoreType.DMA,
        ],
        compiler_params=pltpu.CompilerParams(use_tc_tiling_on_sc=False),
    )


def kernel(x, edge_index, batch,
           Wq1, bq1, Wk1, bk1, Wv1, bv1, Ws1, bs1, Wb1,
           Wq2, bq2, Wk2, bk2, Wv2, bv2, Ws2, bs2, Wb2,
           Wl, bl):
    src = edge_index[0]
    dst = edge_index[1]
    s = 0.125

    w1 = jnp.concatenate([Wq1 * s, Wk1, Wv1, Ws1], axis=1)
    b1 = jnp.concatenate([bq1 * s, bk1, bv1, bs1]).reshape(1, -1)
    w2 = jnp.concatenate([Wq2 * s, Wk2, Wv2, Ws2], axis=1)
    b2 = jnp.concatenate([bq2 * s, bk2, bv2, bs2]).reshape(1, -1)
    wba1 = Wb1[:HC] + Wb1[2 * HC:]
    wbb1 = Wb1[HC:2 * HC] - Wb1[2 * HC:]
    wba2 = Wb2[:HC] + Wb2[2 * HC:]
    wbb2 = Wb2[HC:2 * HC] - Wb2[2 * HC:]

    edge_attention = _make_edge_attention()
    p1 = _project(x, w1, b1, rows=1000)
    q1, kv1, xr1 = p1[:H], p1[H:2 * H], p1[2 * H]
    acc1 = edge_attention(src, dst, q1[0], q1[1], q1[2], kv1[0], kv1[1], kv1[2])
    p2 = _mid(acc1.reshape(2 * H, N, AW), xr1, wba1, wbb1, w2, b2, rows=1000)
    q2, kv2, xr2 = p2[:H], p2[H:2 * H], p2[2 * H]
    acc2 = edge_attention(src, dst, q2[0], q2[1], q2[2], kv2[0], kv2[1], kv2[2])
    logits = _final(acc2.reshape(2 * H, N, AW), xr2, wba2, wbb2,
                    batch.reshape(N, 1), Wl, bl.reshape(1, 1), rows=1000)
    return logits[:, 0]


# speedup vs baseline: 20.8866x; 1.6230x over previous
import functools
import jax
import jax.numpy as jnp
from jax import lax
from jax.experimental import pallas as pl
from jax.experimental.pallas import tpu as pltpu
from jax.experimental.pallas import tpu_sc as plsc

N = 10000
E = 320000
DIN = 128
H = 3
C = 64
HC = H * C
G = 64
AW = C + 16

NCORE = 2
NSUB = 16
NTILE = NCORE * NSUB
EPT = E // NTILE
BLK = 80
NBLK = EPT // BLK
RSTRIDE = 624
RSPAN = 640
ZR = 128


def _split_heads(p):
    qs = [p[:, h * C:(h + 1) * C] for h in range(H)]
    kvs = [jnp.concatenate([p[:, HC + h * C:HC + (h + 1) * C],
                            p[:, 2 * HC + h * C:2 * HC + (h + 1) * C]], axis=1)
           for h in range(H)]
    return qs, kvs, p[:, 3 * HC:]


def _proj_body(x_ref, w_ref, b_ref, *o_refs):
    p = jnp.dot(x_ref[...], w_ref[...], preferred_element_type=jnp.float32,
            precision=lax.Precision.HIGHEST)
    p = p + b_ref[...]
    qs, kvs, xr = _split_heads(p)
    for h in range(H):
        o_refs[h][...] = qs[h]
        o_refs[H + h][...] = kvs[h]
    o_refs[2 * H][...] = xr


def _head_out_specs(rows):
    return (
        [pl.BlockSpec((rows, C), lambda i: (i, 0)) for _ in range(H)]
        + [pl.BlockSpec((rows, 2 * C), lambda i: (i, 0)) for _ in range(H)]
        + [pl.BlockSpec((rows, HC), lambda i: (i, 0))]
    )


def _head_out_shapes():
    return (
        [jax.ShapeDtypeStruct((N, C), jnp.float32) for _ in range(H)]
        + [jax.ShapeDtypeStruct((N, 2 * C), jnp.float32) for _ in range(H)]
        + [jax.ShapeDtypeStruct((N, HC), jnp.float32)]
    )


def _project(x, wcat, bcat, rows):
    din = x.shape[1]
    return pl.pallas_call(
        _proj_body,
        grid=(N // rows,),
        in_specs=[
            pl.BlockSpec((rows, din), lambda i: (i, 0)),
            pl.BlockSpec((din, 4 * HC), lambda i: (0, 0)),
            pl.BlockSpec((1, 4 * HC), lambda i: (0, 0)),
        ],
        out_specs=_head_out_specs(rows),
        out_shape=_head_out_shapes(),
    )(x, wcat, bcat)


def _finish_layer(acc6, xr, wba, wbb):
    outs = []
    for h in range(H):
        a = acc6[2 * h] + acc6[2 * h + 1]
        den = a[:, C:C + 1]
        outs.append(a[:, :C] / (den + 1e-16))
    out = jnp.concatenate(outs, axis=1)
    blog = (jnp.dot(out, wba, preferred_element_type=jnp.float32,
            precision=lax.Precision.HIGHEST)
            + jnp.dot(xr, wbb, preferred_element_type=jnp.float32,
            precision=lax.Precision.HIGHEST))
    beta = 1.0 / (1.0 + jnp.exp(-blog))
    z = beta * xr + (1.0 - beta) * out
    return jnp.where(z > 0, z, jnp.exp(z) - 1.0)


def _mid_body(a_ref, xr_ref, wba_ref, wbb_ref, w_ref, b_ref,
              *o_refs):
    h = _finish_layer(a_ref[...], xr_ref[...], wba_ref[...], wbb_ref[...])
    p = jnp.dot(h, w_ref[...], preferred_element_type=jnp.float32,
            precision=lax.Precision.HIGHEST) + b_ref[...]
    qs, kvs, xr = _split_heads(p)
    for i in range(H):
        o_refs[i][...] = qs[i]
        o_refs[H + i][...] = kvs[i]
    o_refs[2 * H][...] = xr


def _mid(acc6, xr, wba, wbb, wcat, bcat, rows):
    return pl.pallas_call(
        _mid_body,
        grid=(N // rows,),
        in_specs=[
            pl.BlockSpec((2 * H, rows, AW), lambda i: (0, i, 0)),
            pl.BlockSpec((rows, HC), lambda i: (i, 0)),
            pl.BlockSpec((HC, 1), lambda i: (0, 0)),
            pl.BlockSpec((HC, 1), lambda i: (0, 0)),
            pl.BlockSpec((HC, 4 * HC), lambda i: (0, 0)),
            pl.BlockSpec((1, 4 * HC), lambda i: (0, 0)),
        ],
        out_specs=_head_out_specs(rows),
        out_shape=_head_out_shapes(),
    )(acc6, xr, wba, wbb, wcat, bcat)


def _final_body(a_ref, xr_ref, wba_ref, wbb_ref, b_ref,
                wl_ref, bl_ref, out_ref, sum_ref, cnt_ref):
    i = pl.program_id(0)
    rows = xr_ref.shape[0]

    @pl.when(i == 0)
    def _():
        sum_ref[...] = jnp.zeros_like(sum_ref)
        cnt_ref[...] = jnp.zeros_like(cnt_ref)

    h = _finish_layer(a_ref[...], xr_ref[...], wba_ref[...], wbb_ref[...])
    bb = b_ref[...]
    mt = (jnp.broadcast_to(bb, (rows, G))
          == lax.broadcasted_iota(jnp.int32, (rows, G), 1)).astype(jnp.float32)
    dn = (((0,), (0,)), ((), ()))
    sum_ref[...] += lax.dot_general(mt, h, dn, preferred_element_type=jnp.float32,
            precision=lax.Precision.HIGHEST)
    cnt_ref[...] += lax.dot_general(mt, jnp.ones((rows, 1), jnp.float32), dn,
                                    preferred_element_type=jnp.float32,
            precision=lax.Precision.HIGHEST)

    @pl.when(i == pl.num_programs(0) - 1)
    def _():
        t = jnp.dot(sum_ref[...], wl_ref[...], preferred_element_type=jnp.float32,
            precision=lax.Precision.HIGHEST)
        out_ref[...] = t / jnp.maximum(cnt_ref[...], 1.0) + bl_ref[...]


def _final(acc6, xr, wba, wbb, batch2d, wl, blb, rows):
    return pl.pallas_call(
        _final_body,
        grid=(N // rows,),
        in_specs=[
            pl.BlockSpec((2 * H, rows, AW), lambda i: (0, i, 0)),
            pl.BlockSpec((rows, HC), lambda i: (i, 0)),
            pl.BlockSpec((HC, 1), lambda i: (0, 0)),
            pl.BlockSpec((HC, 1), lambda i: (0, 0)),
            pl.BlockSpec((rows, 1), lambda i: (i, 0)),
            pl.BlockSpec((HC, 1), lambda i: (0, 0)),
            pl.BlockSpec((1, 1), lambda i: (0, 0)),
        ],
        out_specs=pl.BlockSpec((G, 1), lambda i: (0, 0)),
        out_shape=jax.ShapeDtypeStruct((G, 1), jnp.float32),
        scratch_shapes=[
            pltpu.VMEM((G, HC), jnp.float32),
            pltpu.VMEM((G, 1), jnp.float32),
        ],
    )(acc6, xr, wba, wbb, batch2d, wl, blb)


def _edge_body(src_hbm, dst_hbm, q0_hbm, q1_hbm, q2_hbm,
               kv0_hbm, kv1_hbm, kv2_hbm, out_hbm,
               acc_sh, sall, dall, qrows2, kvrows2, msg2, zbuf,
               gq0, gq1, gkv0, gkv1, ss0, ss1):
    cid = lax.axis_index("c")
    sid = lax.axis_index("s")
    wid = sid * NCORE + cid
    gq = (gq0, gq1)
    gkv = (gkv0, gkv1)
    ss = (ss0, ss1)

    lane = lax.broadcasted_iota(jnp.int32, (16,), 0)
    zero16 = jnp.zeros((16,), jnp.float32)

    pltpu.sync_copy(src_hbm.at[pl.ds(wid * NBLK, NBLK)], sall)
    pltpu.sync_copy(dst_hbm.at[pl.ds(wid * NBLK, NBLK)], dall)

    def _zrow(r, _):
        for j in range(AW // 16):
            zbuf[r, pl.ds(j * 16, 16)] = zero16
        return 0
    lax.fori_loop(0, ZR, _zrow, 0)

    def _issue_gather(q_hbm, kv_hbm, i, b):
        pltpu.async_copy(q_hbm.at[dall.at[i]], qrows2.at[b], gq[b])
        pltpu.async_copy(kv_hbm.at[sall.at[i]], kvrows2.at[b], gkv[b])

    def _drain_gather(q_hbm, kv_hbm, b):
        pltpu.make_async_copy(q_hbm.at[dall.at[0]], qrows2.at[b], gq[b]).wait()
        pltpu.make_async_copy(kv_hbm.at[sall.at[0]], kvrows2.at[b],
                              gkv[b]).wait()

    def _drain_scatter(b):
        pltpu.make_async_copy(msg2.at[b], acc_sh.at[dall.at[0]], ss[b]).wait()

    def _compute(b):
        def _edge(e, _):
            acc = qrows2[b, e, pl.ds(0, 16)] * kvrows2[b, e, pl.ds(0, 16)]
            for j in range(1, C // 16):
                acc = acc + (qrows2[b, e, pl.ds(j * 16, 16)]
                             * kvrows2[b, e, pl.ds(j * 16, 16)])
            for sh in (8, 4, 2, 1):
                acc = acc + jnp.take(acc, lane ^ sh)
            ex = jnp.exp(acc)
            for j in range(C // 16):
                msg2[b, e, pl.ds(j * 16, 16)] = (
                    kvrows2[b, e, pl.ds(C + j * 16, 16)] * ex)
            msg2[b, e, pl.ds(C, 16)] = jnp.where(lane == 0, ex, zero16)
            return 0
        lax.fori_loop(0, BLK, _edge, 0)

    for h, (q_hbm, kv_hbm) in enumerate(
            ((q0_hbm, kv0_hbm), (q1_hbm, kv1_hbm), (q2_hbm, kv2_hbm))):
        for t in range(RSPAN // ZR):
            pltpu.sync_copy(zbuf, acc_sh.at[pl.ds(sid * RSTRIDE + t * ZR, ZR)])
        plsc.subcore_barrier()

        _issue_gather(q_hbm, kv_hbm, 0, 0)

        def _pair(g, _):
            for b in (0, 1):
                i = 2 * g + b
                _issue_gather(q_hbm, kv_hbm, i + 1, 1 - b)
                _drain_gather(q_hbm, kv_hbm, b)

                @pl.when(g > 0)
                def _():
                    _drain_scatter(b)
                _compute(b)
                pltpu.async_copy(msg2.at[b], acc_sh.at[dall.at[i]], ss[b],
                                 add=True)
            return 0
        lax.fori_loop(0, (NBLK - 1) // 2, _pair, 0)

        _drain_gather(q_hbm, kv_hbm, 0)
        _drain_scatter(0)
        _compute(0)
        pltpu.async_copy(msg2.at[0], acc_sh.at[dall.at[NBLK - 1]], ss[0],
                         add=True)
        _drain_scatter(1)
        _drain_scatter(0)

        plsc.subcore_barrier()
        pltpu.sync_copy(acc_sh.at[pl.ds(sid * RSTRIDE, RSPAN)],
                        out_hbm.at[h, cid, pl.ds(sid * RSTRIDE, RSPAN)])
        plsc.subcore_barrier()


@functools.lru_cache(maxsize=None)
def _make_edge_attention():
    return pl.kernel(
        _edge_body,
        out_type=jax.ShapeDtypeStruct((H, NCORE, N, AW), jnp.float32),
        mesh=plsc.VectorSubcoreMesh(core_axis_name="c", subcore_axis_name="s"),
        scratch_types=[
            pltpu.VMEM_SHARED((N, AW), jnp.float32),
            pltpu.VMEM((NBLK, BLK), jnp.int32),
            pltpu.VMEM((NBLK, BLK), jnp.int32),
            pltpu.VMEM((2, BLK, C), jnp.float32),
            pltpu.VMEM((2, BLK, 2 * C), jnp.float32),
            pltpu.VMEM((2, BLK, AW), jnp.float32),
            pltpu.VMEM((ZR, AW), jnp.float32),
            pltpu.SemaphoreType.DMA,
            pltpu.SemaphoreType.DMA,
            pltpu.SemaphoreType.DMA,
            pltpu.SemaphoreType.DMA,
            pltpu.SemaphoreType.DMA,
            pltpu.SemaphoreType.DMA,
        ],
        compiler_params=pltpu.CompilerParams(use_tc_tiling_on_sc=False),
    )


def kernel(x, edge_index, batch,
           Wq1, bq1, Wk1, bk1, Wv1, bv1, Ws1, bs1, Wb1,
           Wq2, bq2, Wk2, bk2, Wv2, bv2, Ws2, bs2, Wb2,
           Wl, bl):
    src = edge_index[0]
    dst = edge_index[1]
    s = 0.125

    w1 = jnp.concatenate([Wq1 * s, Wk1, Wv1, Ws1], axis=1)
    b1 = jnp.concatenate([bq1 * s, bk1, bv1, bs1]).reshape(1, -1)
    w2 = jnp.concatenate([Wq2 * s, Wk2, Wv2, Ws2], axis=1)
    b2 = jnp.concatenate([bq2 * s, bk2, bv2, bs2]).reshape(1, -1)
    wba1 = Wb1[:HC] + Wb1[2 * HC:]
    wbb1 = Wb1[HC:2 * HC] - Wb1[2 * HC:]
    wba2 = Wb2[:HC] + Wb2[2 * HC:]
    wbb2 = Wb2[HC:2 * HC] - Wb2[2 * HC:]

    edge_attention = _make_edge_attention()
    src = src.reshape(E // BLK, BLK)
    dst = dst.reshape(E // BLK, BLK)
    p1 = _project(x, w1, b1, rows=1000)
    q1, kv1, xr1 = p1[:H], p1[H:2 * H], p1[2 * H]
    acc1 = edge_attention(src, dst, q1[0], q1[1], q1[2], kv1[0], kv1[1], kv1[2])
    p2 = _mid(acc1.reshape(2 * H, N, AW), xr1, wba1, wbb1, w2, b2, rows=1000)
    q2, kv2, xr2 = p2[:H], p2[H:2 * H], p2[2 * H]
    acc2 = edge_attention(src, dst, q2[0], q2[1], q2[2], kv2[0], kv2[1], kv2[2])
    logits = _final(acc2.reshape(2 * H, N, AW), xr2, wba2, wbb2,
                    batch.reshape(N, 1), Wl, bl.reshape(1, 1), rows=1000)
    return logits[:, 0]


# speedup vs baseline: 56.9379x; 2.7260x over previous
import functools
import jax
import jax.numpy as jnp
from jax import lax
from jax.experimental import pallas as pl
from jax.experimental.pallas import tpu as pltpu
from jax.experimental.pallas import tpu_sc as plsc

N = 10000
E = 320000
DIN = 128
H = 3
C = 64
HC = H * C
G = 64
AW = C + 16

NCORE = 2
NSUB = 16
NTILE = NCORE * NSUB
EPT = E // NTILE
BLK = 80
NBLK = EPT // BLK
RSTRIDE = 624
RSPAN = 640
ZR = 128


def _split_heads(p):
    qs = [p[:, h * C:(h + 1) * C] for h in range(H)]
    kvs = [jnp.concatenate([p[:, HC + h * C:HC + (h + 1) * C],
                            p[:, 2 * HC + h * C:2 * HC + (h + 1) * C]], axis=1)
           for h in range(H)]
    return qs, kvs, p[:, 3 * HC:]


def _proj_body(x_ref, w_ref, b_ref, *o_refs):
    p = jnp.dot(x_ref[...], w_ref[...], preferred_element_type=jnp.float32,
            precision=lax.Precision.HIGHEST)
    p = p + b_ref[...]
    qs, kvs, xr = _split_heads(p)
    for h in range(H):
        o_refs[h][...] = qs[h]
        o_refs[H + h][...] = kvs[h]
    o_refs[2 * H][...] = xr


def _head_out_specs(rows):
    return (
        [pl.BlockSpec((rows, C), lambda i: (i, 0)) for _ in range(H)]
        + [pl.BlockSpec((rows, 2 * C), lambda i: (i, 0)) for _ in range(H)]
        + [pl.BlockSpec((rows, HC), lambda i: (i, 0))]
    )


def _head_out_shapes():
    return (
        [jax.ShapeDtypeStruct((N, C), jnp.float32) for _ in range(H)]
        + [jax.ShapeDtypeStruct((N, 2 * C), jnp.float32) for _ in range(H)]
        + [jax.ShapeDtypeStruct((N, HC), jnp.float32)]
    )


def _project(x, wcat, bcat, rows):
    din = x.shape[1]
    return pl.pallas_call(
        _proj_body,
        grid=(N // rows,),
        in_specs=[
            pl.BlockSpec((rows, din), lambda i: (i, 0)),
            pl.BlockSpec((din, 4 * HC), lambda i: (0, 0)),
            pl.BlockSpec((1, 4 * HC), lambda i: (0, 0)),
        ],
        out_specs=_head_out_specs(rows),
        out_shape=_head_out_shapes(),
    )(x, wcat, bcat)


def _finish_layer(acc6, xr, wba, wbb):
    outs = []
    for h in range(H):
        a = acc6[2 * h] + acc6[2 * h + 1]
        den = a[:, C:C + 1]
        outs.append(a[:, :C] / (den + 1e-16))
    out = jnp.concatenate(outs, axis=1)
    blog = (jnp.dot(out, wba, preferred_element_type=jnp.float32,
            precision=lax.Precision.HIGHEST)
            + jnp.dot(xr, wbb, preferred_element_type=jnp.float32,
            precision=lax.Precision.HIGHEST))
    beta = 1.0 / (1.0 + jnp.exp(-blog))
    z = beta * xr + (1.0 - beta) * out
    return jnp.where(z > 0, z, jnp.exp(z) - 1.0)


def _mid_body(a_ref, xr_ref, wba_ref, wbb_ref, w_ref, b_ref,
              *o_refs):
    h = _finish_layer(a_ref[...], xr_ref[...], wba_ref[...], wbb_ref[...])
    p = jnp.dot(h, w_ref[...], preferred_element_type=jnp.float32,
            precision=lax.Precision.HIGHEST) + b_ref[...]
    qs, kvs, xr = _split_heads(p)
    for i in range(H):
        o_refs[i][...] = qs[i]
        o_refs[H + i][...] = kvs[i]
    o_refs[2 * H][...] = xr


def _mid(acc6, xr, wba, wbb, wcat, bcat, rows):
    return pl.pallas_call(
        _mid_body,
        grid=(N // rows,),
        in_specs=[
            pl.BlockSpec((2 * H, rows, AW), lambda i: (0, i, 0)),
            pl.BlockSpec((rows, HC), lambda i: (i, 0)),
            pl.BlockSpec((HC, 1), lambda i: (0, 0)),
            pl.BlockSpec((HC, 1), lambda i: (0, 0)),
            pl.BlockSpec((HC, 4 * HC), lambda i: (0, 0)),
            pl.BlockSpec((1, 4 * HC), lambda i: (0, 0)),
        ],
        out_specs=_head_out_specs(rows),
        out_shape=_head_out_shapes(),
    )(acc6, xr, wba, wbb, wcat, bcat)


def _final_body(a_ref, xr_ref, wba_ref, wbb_ref, b_ref,
                wl_ref, bl_ref, out_ref, sum_ref, cnt_ref):
    i = pl.program_id(0)
    rows = xr_ref.shape[0]

    @pl.when(i == 0)
    def _():
        sum_ref[...] = jnp.zeros_like(sum_ref)
        cnt_ref[...] = jnp.zeros_like(cnt_ref)

    h = _finish_layer(a_ref[...], xr_ref[...], wba_ref[...], wbb_ref[...])
    bb = b_ref[...]
    mt = (jnp.broadcast_to(bb, (rows, G))
          == lax.broadcasted_iota(jnp.int32, (rows, G), 1)).astype(jnp.float32)
    dn = (((0,), (0,)), ((), ()))
    sum_ref[...] += lax.dot_general(mt, h, dn, preferred_element_type=jnp.float32,
            precision=lax.Precision.HIGHEST)
    cnt_ref[...] += lax.dot_general(mt, jnp.ones((rows, 1), jnp.float32), dn,
                                    preferred_element_type=jnp.float32,
            precision=lax.Precision.HIGHEST)

    @pl.when(i == pl.num_programs(0) - 1)
    def _():
        t = jnp.dot(sum_ref[...], wl_ref[...], preferred_element_type=jnp.float32,
            precision=lax.Precision.HIGHEST)
        out_ref[...] = t / jnp.maximum(cnt_ref[...], 1.0) + bl_ref[...]


def _final(acc6, xr, wba, wbb, batch2d, wl, blb, rows):
    return pl.pallas_call(
        _final_body,
        grid=(N // rows,),
        in_specs=[
            pl.BlockSpec((2 * H, rows, AW), lambda i: (0, i, 0)),
            pl.BlockSpec((rows, HC), lambda i: (i, 0)),
            pl.BlockSpec((HC, 1), lambda i: (0, 0)),
            pl.BlockSpec((HC, 1), lambda i: (0, 0)),
            pl.BlockSpec((rows, 1), lambda i: (i, 0)),
            pl.BlockSpec((HC, 1), lambda i: (0, 0)),
            pl.BlockSpec((1, 1), lambda i: (0, 0)),
        ],
        out_specs=pl.BlockSpec((G, 1), lambda i: (0, 0)),
        out_shape=jax.ShapeDtypeStruct((G, 1), jnp.float32),
        scratch_shapes=[
            pltpu.VMEM((G, HC), jnp.float32),
            pltpu.VMEM((G, 1), jnp.float32),
        ],
    )(acc6, xr, wba, wbb, batch2d, wl, blb)


def _edge_body(src_hbm, dst_hbm, q0_hbm, q1_hbm, q2_hbm,
               kv0_hbm, kv1_hbm, kv2_hbm, out_hbm,
               acc_sh, sall, dall, qrows2, kvrows2, msg2, zbuf,
               gq0, gq1, gkv0, gkv1, ss0, ss1):
    cid = lax.axis_index("c")
    sid = lax.axis_index("s")
    wid = sid * NCORE + cid
    gq = (gq0, gq1)
    gkv = (gkv0, gkv1)
    ss = (ss0, ss1)

    lane = lax.broadcasted_iota(jnp.int32, (16,), 0)
    zero16 = jnp.zeros((16,), jnp.float32)

    pltpu.sync_copy(src_hbm.at[pl.ds(wid * NBLK, NBLK)], sall)
    pltpu.sync_copy(dst_hbm.at[pl.ds(wid * NBLK, NBLK)], dall)

    def _zrow(r, _):
        for j in range(AW // 16):
            zbuf[r, pl.ds(j * 16, 16)] = zero16
        return 0
    lax.fori_loop(0, ZR, _zrow, 0)

    def _issue_gather(q_hbm, kv_hbm, i, b):
        pltpu.async_copy(q_hbm.at[dall.at[i]], qrows2.at[b], gq[b])
        pltpu.async_copy(kv_hbm.at[sall.at[i]], kvrows2.at[b], gkv[b])

    def _drain_gather(q_hbm, kv_hbm, b):
        pltpu.make_async_copy(q_hbm.at[dall.at[0]], qrows2.at[b], gq[b]).wait()
        pltpu.make_async_copy(kv_hbm.at[sall.at[0]], kvrows2.at[b],
                              gkv[b]).wait()

    def _drain_scatter(b):
        pltpu.make_async_copy(msg2.at[b], acc_sh.at[dall.at[0]], ss[b]).wait()

    def _compute(b):
        @functools.partial(plsc.parallel_loop, 0, BLK, unroll=4)
        def _edge(e):
            acc = qrows2[b, e, pl.ds(0, 16)] * kvrows2[b, e, pl.ds(0, 16)]
            for j in range(1, C // 16):
                acc = acc + (qrows2[b, e, pl.ds(j * 16, 16)]
                             * kvrows2[b, e, pl.ds(j * 16, 16)])
            for sh in (8, 4, 2, 1):
                acc = acc + jnp.take(acc, lane ^ sh)
            ex = jnp.exp(acc)
            for j in range(C // 16):
                msg2[b, e, pl.ds(j * 16, 16)] = (
                    kvrows2[b, e, pl.ds(C + j * 16, 16)] * ex)
            msg2[b, e, pl.ds(C, 16)] = jnp.where(lane == 0, ex, zero16)

    for h, (q_hbm, kv_hbm) in enumerate(
            ((q0_hbm, kv0_hbm), (q1_hbm, kv1_hbm), (q2_hbm, kv2_hbm))):
        for t in range(RSPAN // ZR):
            pltpu.sync_copy(zbuf, acc_sh.at[pl.ds(sid * RSTRIDE + t * ZR, ZR)])
        plsc.subcore_barrier()

        _issue_gather(q_hbm, kv_hbm, 0, 0)

        def _pair(g, _):
            for b in (0, 1):
                i = 2 * g + b
                _issue_gather(q_hbm, kv_hbm, i + 1, 1 - b)
                _drain_gather(q_hbm, kv_hbm, b)

                @pl.when(g > 0)
                def _():
                    _drain_scatter(b)
                _compute(b)
                pltpu.async_copy(msg2.at[b], acc_sh.at[dall.at[i]], ss[b],
                                 add=True)
            return 0
        lax.fori_loop(0, (NBLK - 1) // 2, _pair, 0)

        _drain_gather(q_hbm, kv_hbm, 0)
        _drain_scatter(0)
        _compute(0)
        pltpu.async_copy(msg2.at[0], acc_sh.at[dall.at[NBLK - 1]], ss[0],
                         add=True)
        _drain_scatter(1)
        _drain_scatter(0)

        plsc.subcore_barrier()
        pltpu.sync_copy(acc_sh.at[pl.ds(sid * RSTRIDE, RSPAN)],
                        out_hbm.at[h, cid, pl.ds(sid * RSTRIDE, RSPAN)])
        plsc.subcore_barrier()


@functools.lru_cache(maxsize=None)
def _make_edge_attention():
    return pl.kernel(
        _edge_body,
        out_type=jax.ShapeDtypeStruct((H, NCORE, N, AW), jnp.float32),
        mesh=plsc.VectorSubcoreMesh(core_axis_name="c", subcore_axis_name="s"),
        scratch_types=[
            pltpu.VMEM_SHARED((N, AW), jnp.float32),
            pltpu.VMEM((NBLK, BLK), jnp.int32),
            pltpu.VMEM((NBLK, BLK), jnp.int32),
            pltpu.VMEM((2, BLK, C), jnp.float32),
            pltpu.VMEM((2, BLK, 2 * C), jnp.float32),
            pltpu.VMEM((2, BLK, AW), jnp.float32),
            pltpu.VMEM((ZR, AW), jnp.float32),
            pltpu.SemaphoreType.DMA,
            pltpu.SemaphoreType.DMA,
            pltpu.SemaphoreType.DMA,
            pltpu.SemaphoreType.DMA,
            pltpu.SemaphoreType.DMA,
            pltpu.SemaphoreType.DMA,
        ],
        compiler_params=pltpu.CompilerParams(use_tc_tiling_on_sc=False),
    )


def kernel(x, edge_index, batch,
           Wq1, bq1, Wk1, bk1, Wv1, bv1, Ws1, bs1, Wb1,
           Wq2, bq2, Wk2, bk2, Wv2, bv2, Ws2, bs2, Wb2,
           Wl, bl):
    src = edge_index[0]
    dst = edge_index[1]
    s = 0.125

    w1 = jnp.concatenate([Wq1 * s, Wk1, Wv1, Ws1], axis=1)
    b1 = jnp.concatenate([bq1 * s, bk1, bv1, bs1]).reshape(1, -1)
    w2 = jnp.concatenate([Wq2 * s, Wk2, Wv2, Ws2], axis=1)
    b2 = jnp.concatenate([bq2 * s, bk2, bv2, bs2]).reshape(1, -1)
    wba1 = Wb1[:HC] + Wb1[2 * HC:]
    wbb1 = Wb1[HC:2 * HC] - Wb1[2 * HC:]
    wba2 = Wb2[:HC] + Wb2[2 * HC:]
    wbb2 = Wb2[HC:2 * HC] - Wb2[2 * HC:]

    edge_attention = _make_edge_attention()
    src = src.reshape(E // BLK, BLK)
    dst = dst.reshape(E // BLK, BLK)
    p1 = _project(x, w1, b1, rows=1000)
    q1, kv1, xr1 = p1[:H], p1[H:2 * H], p1[2 * H]
    acc1 = edge_attention(src, dst, q1[0], q1[1], q1[2], kv1[0], kv1[1], kv1[2])
    p2 = _mid(acc1.reshape(2 * H, N, AW), xr1, wba1, wbb1, w2, b2, rows=1000)
    q2, kv2, xr2 = p2[:H], p2[H:2 * H], p2[2 * H]
    acc2 = edge_attention(src, dst, q2[0], q2[1], q2[2], kv2[0], kv2[1], kv2[2])
    logits = _final(acc2.reshape(2 * H, N, AW), xr2, wba2, wbb2,
                    batch.reshape(N, 1), Wl, bl.reshape(1, 1), rows=1000)
    return logits[:, 0]
